# Initial kernel scaffold; baseline (speedup 1.0000x reference)
import functools

import jax
import jax.numpy as jnp
from jax import lax
from jax.experimental import pallas as pl
from jax.experimental.pallas import tpu as pltpu
from jax.experimental.pallas import tpu_sc as plsc

L = 16
D = 32
V = 1000
NC, NS, LANES = 2, 16, 16
NW = NC * NS
CHUNK = 256


def _make_sc_call(n_words):
    wpt = n_words // NW
    n_chunks = wpt // CHUNK
    mesh = plsc.VectorSubcoreMesh(core_axis_name="c", subcore_axis_name="s")

    @functools.partial(
        pl.kernel,
        out_type=jax.ShapeDtypeStruct((n_words * D,), jnp.float32),
        mesh=mesh,
        scratch_types=[
            pltpu.VMEM((V * D,), jnp.float32),
            pltpu.VMEM((CHUNK * L,), jnp.int32),
            pltpu.VMEM((CHUNK * D,), jnp.float32),
        ],
    )
    def sc_kernel(ids_hbm, table_hbm, out_hbm, table_v, ids_v, out_v):
        wid = lax.axis_index("s") * NC + lax.axis_index("c")
        base_word = wid * wpt

        pltpu.sync_copy(table_hbm, table_v)

        iota = lax.iota(jnp.int32, LANES)
        iota_l = iota * L
        iota_d = iota * D

        def chunk_body(ci, carry):
            word0 = base_word + ci * CHUNK
            pltpu.sync_copy(ids_hbm.at[pl.ds(word0 * L, CHUNK * L)], ids_v)

            def group_body(g, carry2):
                gbase = g * (LANES * L)
                rows = []
                for l in range(L):
                    r = plsc.load_gather(ids_v, [gbase + l + iota_l])
                    rows.append(r * D)
                obase = g * (LANES * D)
                for d in range(D):
                    acc = plsc.load_gather(table_v, [rows[0] + d])
                    for l in range(1, L):
                        acc = acc + plsc.load_gather(table_v, [rows[l] + d])
                    plsc.store_scatter(out_v, [obase + d + iota_d], acc)
                return carry2

            lax.fori_loop(0, CHUNK // LANES, group_body, 0)
            pltpu.sync_copy(out_v, out_hbm.at[pl.ds(word0 * D, CHUNK * D)])
            return carry

        lax.fori_loop(0, n_chunks, chunk_body, 0)

    return sc_kernel


def kernel(token_ids, table):
    b, w, l = token_ids.shape
    n_words = b * w
    ids_flat = token_ids.astype(jnp.int32).reshape(n_words * L)
    table_flat = table.reshape(V * D)
    out = _make_sc_call(n_words)(ids_flat, table_flat)
    return out.reshape(b, w, D)


# speedup vs baseline: 5.8102x; 5.8102x over previous
import functools

import jax
import jax.numpy as jnp
from jax import lax
from jax.experimental import pallas as pl
from jax.experimental.pallas import tpu as pltpu
from jax.experimental.pallas import tpu_sc as plsc

L = 16
D = 32
V = 1000
NC, NS, LANES = 2, 16, 16
NW = NC * NS
CHUNK = 256


def _make_sc_call(n_words):
    wpt = n_words // NW
    n_chunks = wpt // CHUNK
    mesh = plsc.VectorSubcoreMesh(core_axis_name="c", subcore_axis_name="s")

    @functools.partial(
        pl.kernel,
        out_type=jax.ShapeDtypeStruct((n_words * D,), jnp.float32),
        mesh=mesh,
        scratch_types=[
            pltpu.VMEM((V * D,), jnp.float32),
            pltpu.VMEM((CHUNK * L,), jnp.int32),
            pltpu.VMEM((CHUNK * D,), jnp.float32),
        ],
        compiler_params=pltpu.CompilerParams(needs_layout_passes=False),
    )
    def sc_kernel(ids_hbm, table_hbm, out_hbm, table_v, ids_v, out_v):
        wid = lax.axis_index("s") * NC + lax.axis_index("c")
        base_word = wid * wpt

        pltpu.sync_copy(table_hbm, table_v)

        iota = lax.iota(jnp.int32, LANES)
        iota_l = iota * L
        iota_d = iota * D

        def chunk_body(ci, carry):
            word0 = base_word + ci * CHUNK
            pltpu.sync_copy(ids_hbm.at[pl.ds(word0 * L, CHUNK * L)], ids_v)

            def group_body(g, carry2):
                gbase = g * (LANES * L)
                rows = []
                for l in range(L):
                    r = plsc.load_gather(ids_v, [gbase + l + iota_l])
                    rows.append(r * D)
                obase = g * (LANES * D)
                for d in range(D):
                    acc = plsc.load_gather(table_v, [rows[0] + d])
                    for l in range(1, L):
                        acc = acc + plsc.load_gather(table_v, [rows[l] + d])
                    plsc.store_scatter(out_v, [obase + d + iota_d], acc)
                return carry2

            lax.fori_loop(0, CHUNK // LANES, group_body, 0)
            pltpu.sync_copy(out_v, out_hbm.at[pl.ds(word0 * D, CHUNK * D)])
            return carry

        lax.fori_loop(0, n_chunks, chunk_body, 0)

    return sc_kernel


def kernel(token_ids, table):
    b, w, l = token_ids.shape
    n_words = b * w
    ids_flat = token_ids.astype(jnp.int32).reshape(n_words * L)
    table_flat = table.reshape(V * D)
    out = _make_sc_call(n_words)(ids_flat, table_flat)
    return out.reshape(b, w, D)
